# baseline (speedup 1.0000x reference)
import functools

import jax
import jax.numpy as jnp
from jax import lax
from jax.experimental import pallas as pl
from jax.experimental.pallas import tpu as pltpu
from jax.experimental.pallas import tpu_sc as plsc

N = 10000
E = 320000
G = 64
NC, NS = 2, 16
NW = NC * NS
BLK = 128
BPW = 80
E_PAD = NW * BPW * BLK
N_ACC = 10112
RPT = N_ACC // NS


def _sc_mesh():
    return plsc.VectorSubcoreMesh(core_axis_name="c", subcore_axis_name="s")


@functools.cache
def _sc_spmm_kernel(F, BPW0):
    NB = 8
    LAI = 4
    LAG = 3
    ZR = 158
    BPW1 = 2 * BPW - BPW0
    NOUT = NC if BPW1 else 1

    @functools.partial(
        pl.kernel,
        out_type=jax.ShapeDtypeStruct((NOUT * N_ACC, F), jnp.float32),
        mesh=_sc_mesh(),
        scratch_types=(
            [pltpu.VMEM((BLK,), jnp.int32) for _ in range(NB)]
            + [pltpu.VMEM((BLK,), jnp.int32) for _ in range(NB)]
            + [pltpu.VMEM((BLK, F), jnp.float32) for _ in range(NB)]
            + [pltpu.VMEM((ZR, F), jnp.float32),
               pltpu.VMEM_SHARED((N_ACC, F), jnp.float32)]
            + [pltpu.SemaphoreType.DMA for _ in range(3 * NB)]
        ),
        compiler_params=pltpu.CompilerParams(use_tc_tiling_on_sc=False),
    )
    def k(h_hbm, src_hbm, dst_hbm, out_hbm, *refs):
        sidx = refs[0:NB]
        didx = refs[NB:2 * NB]
        rows = refs[2 * NB:3 * NB]
        zbuf_v = refs[3 * NB]
        acc_sh = refs[3 * NB + 1]
        sem_i = refs[3 * NB + 2:3 * NB + 2 + NB]
        sem_g = refs[3 * NB + 2 + NB:3 * NB + 2 + 2 * NB]
        sem_s = refs[3 * NB + 2 + 2 * NB:3 * NB + 2 + 3 * NB]
        c = lax.axis_index("c")
        s = lax.axis_index("s")
        z16 = jnp.zeros((16,), jnp.float32)
        row0 = s * RPT

        def do_zero():
            def zrow(r, carry):
                for j in range(F // 16):
                    zbuf_v[r, pl.ds(j * 16, 16)] = z16
                return carry

            lax.fori_loop(0, ZR, zrow, 0)
            for t in range(RPT // ZR):
                pltpu.sync_copy(zbuf_v, acc_sh.at[pl.ds(row0 + t * ZR, ZR)])

        def do_writeout(base):
            for t in range(RPT // ZR):
                pltpu.sync_copy(acc_sh.at[pl.ds(row0 + t * ZR, ZR)], zbuf_v)
                pltpu.sync_copy(zbuf_v,
                                out_hbm.at[pl.ds(base + row0 + t * ZR, ZR)])

        def pipeline(nblk, blk0):
            def fire_idx(j):
                b = j % NB
                off = pl.multiple_of((blk0 + j) * BLK, BLK)
                return (
                    pltpu.async_copy(src_hbm.at[pl.ds(off, BLK)], sidx[b],
                                     sem_i[b]),
                    pltpu.async_copy(dst_hbm.at[pl.ds(off, BLK)], didx[b],
                                     sem_i[b]),
                )

            def fire_gather(j):
                b = j % NB
                return pltpu.async_copy(h_hbm.at[sidx[b]], rows[b], sem_g[b])

            idxd = [None] * nblk
            gat = [None] * nblk
            scat = [None] * nblk
            for j in range(min(LAI, nblk)):
                idxd[j] = fire_idx(j)
            for j in range(min(LAG, nblk)):
                for d in idxd[j]:
                    d.wait()
                gat[j] = fire_gather(j)
            for j in range(nblk):
                b = j % NB
                if j + LAI < nblk:
                    if j + LAI >= NB:
                        scat[j + LAI - NB].wait()
                    idxd[j + LAI] = fire_idx(j + LAI)
                if j + LAG < nblk:
                    for d in idxd[j + LAG]:
                        d.wait()
                    gat[j + LAG] = fire_gather(j + LAG)
                gat[j].wait()
                scat[j] = pltpu.async_copy(rows[b], acc_sh.at[didx[b]],
                                           sem_s[b], add=True)
            for j in range(max(nblk - NB, 0), nblk):
                scat[j].wait()

        if BPW1:
            do_zero()
            plsc.subcore_barrier()
            pl.when(c == 0)(lambda: pipeline(BPW0, s * BPW0))
            pl.when(c == 1)(lambda: pipeline(BPW1, NS * BPW0 + s * BPW1))
            plsc.subcore_barrier()
            do_writeout(c * N_ACC)
        else:
            pl.when(c == 0)(do_zero)
            plsc.subcore_barrier()
            pl.when(c == 0)(lambda: pipeline(BPW0, s * BPW0))
            plsc.subcore_barrier()
            pl.when(c == 0)(lambda: do_writeout(0))

    return k


def _sc_spmm(h, src1d, dst1d, F):
    return _sc_spmm_kernel(F, 2 * BPW)(h, src1d, dst1d)


def _sc_degree(dst1d):
    FD = 16

    NB = 4

    @functools.partial(
        pl.kernel,
        out_type=jax.ShapeDtypeStruct((NC * N_ACC, FD), jnp.float32),
        mesh=_sc_mesh(),
        scratch_types=(
            [pltpu.VMEM((BLK,), jnp.int32) for _ in range(NB)]
            + [pltpu.VMEM((BLK, FD), jnp.float32),
               pltpu.VMEM((RPT, FD), jnp.float32),
               pltpu.VMEM_SHARED((N_ACC, FD), jnp.float32)]
            + [pltpu.SemaphoreType.DMA for _ in range(2 * NB)]
        ),
        compiler_params=pltpu.CompilerParams(use_tc_tiling_on_sc=False),
    )
    def k(dst_hbm, out_hbm, *refs):
        didx = refs[0:NB]
        ones_v = refs[NB]
        zbuf_v = refs[NB + 1]
        acc_sh = refs[NB + 2]
        sem_i = refs[NB + 3:NB + 3 + NB]
        sem_s = refs[NB + 3 + NB:NB + 3 + 2 * NB]
        c = lax.axis_index("c")
        s = lax.axis_index("s")
        w = c * NS + s
        z16 = jnp.zeros((16,), jnp.float32)
        o16 = jnp.ones((16,), jnp.float32)

        def zrow(r, carry):
            zbuf_v[r, pl.ds(0, 16)] = z16
            return carry

        lax.fori_loop(0, RPT, zrow, 0)

        def orow(r, carry):
            ones_v[r, pl.ds(0, 16)] = o16
            return carry

        lax.fori_loop(0, BLK, orow, 0)
        row0 = s * RPT
        pltpu.sync_copy(zbuf_v, acc_sh.at[pl.ds(row0, RPT)])
        plsc.subcore_barrier()

        def fire_idx(j):
            b = j % NB
            off = pl.multiple_of((w * BPW + j) * BLK, BLK)
            return pltpu.async_copy(dst_hbm.at[pl.ds(off, BLK)], didx[b],
                                    sem_i[b])

        idxd = [None] * BPW
        scat = [None] * BPW
        idxd[0] = fire_idx(0)
        for j in range(BPW):
            b = j % NB
            if j + 1 < BPW:
                if j + 1 >= NB:
                    scat[j + 1 - NB].wait()
                idxd[j + 1] = fire_idx(j + 1)
            idxd[j].wait()
            scat[j] = pltpu.async_copy(ones_v, acc_sh.at[didx[b]],
                                       sem_s[b], add=True)
        for j in range(BPW - NB, BPW):
            scat[j].wait()
        plsc.subcore_barrier()
        pltpu.sync_copy(acc_sh.at[pl.ds(row0, RPT)], zbuf_v)
        pltpu.sync_copy(zbuf_v, out_hbm.at[pl.ds(c * N_ACC + row0, RPT)])

    return k(dst1d).reshape(NC, N_ACC, FD)


def _tc_prescale1(degp, x, W1):
    def body(degp_ref, x_ref, w1_ref, h1p_ref, dinv_ref):
        deg = degp_ref[0, :N, 0:1] + degp_ref[1, :N, 0:1] + 1.0
        dinv = lax.rsqrt(deg)
        dinv_ref[...] = dinv
        h1p_ref[...] = jnp.dot(x_ref[...], w1_ref[...],
                               preferred_element_type=jnp.float32) * dinv

    return pl.pallas_call(
        body,
        out_shape=[jax.ShapeDtypeStruct((N, 32), jnp.float32),
                   jax.ShapeDtypeStruct((N, 1), jnp.float32)],
    )(degp, x, W1)


def _tc_combine2(s1, h1p, dinv, W2, b1r):
    def body(s1_ref, h1p_ref, dinv_ref, w2_ref, b1_ref, h2p_ref):
        h1 = (s1_ref[:N, :] + h1p_ref[...]) * dinv_ref[...] + b1_ref[...]
        h2p_ref[...] = jnp.dot(h1, w2_ref[...],
                               preferred_element_type=jnp.float32) * dinv_ref[...]

    return pl.pallas_call(
        body,
        out_shape=jax.ShapeDtypeStruct((N, 64), jnp.float32),
    )(s1, h1p, dinv, W2, b1r)


def _tc_head(s2, h2p, dinv, b2r, batch_row, fc1_W, fc1_br, bn_gr, bn_br,
             fc2_W, fc2_br):
    def body(s2_ref, h2p_ref, dinv_ref, b2_ref, bat_ref, w1_ref, bb1_ref,
             g_ref, bgb_ref, w2_ref, bb2_ref, out_ref):
        h2 = (s2_ref[:N, :] + h2p_ref[...]) * dinv_ref[...] + b2_ref[...]
        gids = lax.broadcasted_iota(jnp.int32, (G, N), 0)
        oh = (gids == bat_ref[...]).astype(jnp.float32)
        sums = lax.dot_general(oh, h2, (((1,), (0,)), ((), ())),
                               preferred_element_type=jnp.float32)
        cnts = jnp.sum(oh, axis=1, keepdims=True)
        g = sums / jnp.maximum(cnts, 1.0)
        z = jnp.dot(g, w1_ref[...], preferred_element_type=jnp.float32) \
            + bb1_ref[...]
        z = jnp.maximum(z, 0.0)
        mu = jnp.mean(z, axis=0, keepdims=True)
        var = jnp.mean((z - mu) ** 2, axis=0, keepdims=True)
        z = (z - mu) * lax.rsqrt(var + 1e-5) * g_ref[...] + bgb_ref[...]
        o = jnp.dot(z, w2_ref[...], preferred_element_type=jnp.float32) \
            + bb2_ref[...]
        out_ref[...] = jnp.maximum(o, 0.0) + jnp.log1p(jnp.exp(-jnp.abs(o)))

    return pl.pallas_call(
        body,
        out_shape=jax.ShapeDtypeStruct((G, 1), jnp.float32),
    )(s2, h2p, dinv, b2r, batch_row, fc1_W, fc1_br, bn_gr, bn_br, fc2_W,
      fc2_br)


def kernel(x, W1, b1, W2, b2, fc1_W, fc1_b, bn_g, bn_b, fc2_W, fc2_b,
           edge_index, batch):
    src = edge_index[0]
    dst = edge_index[1]
    pad = E_PAD - E
    src1d = jnp.concatenate([src, jnp.zeros((pad,), jnp.int32)])
    dst1d = jnp.concatenate([dst, jnp.full((pad,), N, jnp.int32)])

    degp = _sc_degree(dst1d)
    h1p, dinv = _tc_prescale1(degp, x, W1)
    s1 = _sc_spmm(h1p, src1d, dst1d, 32)
    h2p = _tc_combine2(s1, h1p, dinv, W2, b1.reshape(1, 32))
    s2 = _sc_spmm(h2p, src1d, dst1d, 64)
    out = _tc_head(s2, h2p, dinv, b2.reshape(1, 64), batch.reshape(1, N),
                   fc1_W, fc1_b.reshape(1, 32), bn_g.reshape(1, 32),
                   bn_b.reshape(1, 32), fc2_W, fc2_b.reshape(1, 1))
    return out


# speedup vs baseline: 1.3103x; 1.3103x over previous
import functools

import jax
import jax.numpy as jnp
from jax import lax
from jax.experimental import pallas as pl
from jax.experimental.pallas import tpu as pltpu
from jax.experimental.pallas import tpu_sc as plsc

N = 10000
E = 320000
G = 64
NC, NS = 2, 16
NW = NC * NS
BLK = 128
BPW = 80
E_PAD = NW * BPW * BLK
N_ACC = 10112
RPT = N_ACC // NS


def _sc_mesh():
    return plsc.VectorSubcoreMesh(core_axis_name="c", subcore_axis_name="s")


@functools.cache
def _sc_spmm_kernel(F, blk, bpw0, NB, LAI, LAG):
    ZR = 158
    tot_bpw = E_PAD // (NW * blk)
    BPW0 = bpw0
    BPW1 = 2 * tot_bpw - BPW0
    NOUT = NC if BPW1 else 1

    @functools.partial(
        pl.kernel,
        out_type=jax.ShapeDtypeStruct((NOUT * N_ACC, F), jnp.float32),
        mesh=_sc_mesh(),
        scratch_types=(
            [pltpu.VMEM((blk,), jnp.int32) for _ in range(NB)]
            + [pltpu.VMEM((blk,), jnp.int32) for _ in range(NB)]
            + [pltpu.VMEM((blk, F), jnp.float32) for _ in range(NB)]
            + [pltpu.VMEM((ZR, F), jnp.float32),
               pltpu.VMEM_SHARED((N_ACC, F), jnp.float32)]
            + [pltpu.SemaphoreType.DMA for _ in range(3 * NB)]
        ),
        compiler_params=pltpu.CompilerParams(use_tc_tiling_on_sc=False),
    )
    def k(h_hbm, src_hbm, dst_hbm, out_hbm, *refs):
        sidx = refs[0:NB]
        didx = refs[NB:2 * NB]
        rows = refs[2 * NB:3 * NB]
        zbuf_v = refs[3 * NB]
        acc_sh = refs[3 * NB + 1]
        sem_i = refs[3 * NB + 2:3 * NB + 2 + NB]
        sem_g = refs[3 * NB + 2 + NB:3 * NB + 2 + 2 * NB]
        sem_s = refs[3 * NB + 2 + 2 * NB:3 * NB + 2 + 3 * NB]
        c = lax.axis_index("c")
        s = lax.axis_index("s")
        z16 = jnp.zeros((16,), jnp.float32)
        row0 = s * RPT

        def do_zero():
            def zrow(r, carry):
                for j in range(F // 16):
                    zbuf_v[r, pl.ds(j * 16, 16)] = z16
                return carry

            lax.fori_loop(0, ZR, zrow, 0)
            for t in range(RPT // ZR):
                pltpu.sync_copy(zbuf_v, acc_sh.at[pl.ds(row0 + t * ZR, ZR)])

        def do_writeout(base):
            for t in range(RPT // ZR):
                pltpu.sync_copy(acc_sh.at[pl.ds(row0 + t * ZR, ZR)], zbuf_v)
                pltpu.sync_copy(zbuf_v,
                                out_hbm.at[pl.ds(base + row0 + t * ZR, ZR)])

        def pipeline(nblk, blk0):
            def fire_idx(j):
                b = j % NB
                off = pl.multiple_of((blk0 + j) * blk, blk)
                return (
                    pltpu.async_copy(src_hbm.at[pl.ds(off, blk)], sidx[b],
                                     sem_i[b]),
                    pltpu.async_copy(dst_hbm.at[pl.ds(off, blk)], didx[b],
                                     sem_i[b]),
                )

            def fire_gather(j):
                b = j % NB
                return pltpu.async_copy(h_hbm.at[sidx[b]], rows[b], sem_g[b])

            idxd = [None] * nblk
            gat = [None] * nblk
            scat = [None] * nblk
            for j in range(min(LAI, nblk)):
                idxd[j] = fire_idx(j)
            for j in range(min(LAG, nblk)):
                for d in idxd[j]:
                    d.wait()
                gat[j] = fire_gather(j)
            for j in range(nblk):
                b = j % NB
                if j + LAI < nblk:
                    if j + LAI >= NB:
                        scat[j + LAI - NB].wait()
                    idxd[j + LAI] = fire_idx(j + LAI)
                if j + LAG < nblk:
                    for d in idxd[j + LAG]:
                        d.wait()
                    gat[j + LAG] = fire_gather(j + LAG)
                gat[j].wait()
                scat[j] = pltpu.async_copy(rows[b], acc_sh.at[didx[b]],
                                           sem_s[b], add=True)
            for j in range(max(nblk - NB, 0), nblk):
                scat[j].wait()

        if BPW1:
            do_zero()
            plsc.subcore_barrier()
            pl.when(c == 0)(lambda: pipeline(BPW0, s * BPW0))
            pl.when(c == 1)(lambda: pipeline(BPW1, NS * BPW0 + s * BPW1))
            plsc.subcore_barrier()
            do_writeout(c * N_ACC)
        else:
            pl.when(c == 0)(do_zero)
            plsc.subcore_barrier()
            pl.when(c == 0)(lambda: pipeline(BPW0, s * BPW0))
            plsc.subcore_barrier()
            pl.when(c == 0)(lambda: do_writeout(0))

    return k


def _sc_spmm(h, src1d, dst1d, F, blk, bpw0, nb, lai, lag):
    return _sc_spmm_kernel(F, blk, bpw0, nb, lai, lag)(h, src1d, dst1d)


def _psum(s_ref):
    if s_ref.shape[0] == NC * N_ACC:
        return s_ref[0:N, :] + s_ref[N_ACC:N_ACC + N, :]
    return s_ref[0:N, :]


def _sc_degree(dst1d):
    FD = 16

    NB = 4

    @functools.partial(
        pl.kernel,
        out_type=jax.ShapeDtypeStruct((NC * N_ACC, FD), jnp.float32),
        mesh=_sc_mesh(),
        scratch_types=(
            [pltpu.VMEM((BLK,), jnp.int32) for _ in range(NB)]
            + [pltpu.VMEM((BLK, FD), jnp.float32),
               pltpu.VMEM((RPT, FD), jnp.float32),
               pltpu.VMEM_SHARED((N_ACC, FD), jnp.float32)]
            + [pltpu.SemaphoreType.DMA for _ in range(2 * NB)]
        ),
        compiler_params=pltpu.CompilerParams(use_tc_tiling_on_sc=False),
    )
    def k(dst_hbm, out_hbm, *refs):
        didx = refs[0:NB]
        ones_v = refs[NB]
        zbuf_v = refs[NB + 1]
        acc_sh = refs[NB + 2]
        sem_i = refs[NB + 3:NB + 3 + NB]
        sem_s = refs[NB + 3 + NB:NB + 3 + 2 * NB]
        c = lax.axis_index("c")
        s = lax.axis_index("s")
        w = c * NS + s
        z16 = jnp.zeros((16,), jnp.float32)
        o16 = jnp.ones((16,), jnp.float32)

        def zrow(r, carry):
            zbuf_v[r, pl.ds(0, 16)] = z16
            return carry

        lax.fori_loop(0, RPT, zrow, 0)

        def orow(r, carry):
            ones_v[r, pl.ds(0, 16)] = o16
            return carry

        lax.fori_loop(0, BLK, orow, 0)
        row0 = s * RPT
        pltpu.sync_copy(zbuf_v, acc_sh.at[pl.ds(row0, RPT)])
        plsc.subcore_barrier()

        def fire_idx(j):
            b = j % NB
            off = pl.multiple_of((w * BPW + j) * BLK, BLK)
            return pltpu.async_copy(dst_hbm.at[pl.ds(off, BLK)], didx[b],
                                    sem_i[b])

        idxd = [None] * BPW
        scat = [None] * BPW
        idxd[0] = fire_idx(0)
        for j in range(BPW):
            b = j % NB
            if j + 1 < BPW:
                if j + 1 >= NB:
                    scat[j + 1 - NB].wait()
                idxd[j + 1] = fire_idx(j + 1)
            idxd[j].wait()
            scat[j] = pltpu.async_copy(ones_v, acc_sh.at[didx[b]],
                                       sem_s[b], add=True)
        for j in range(BPW - NB, BPW):
            scat[j].wait()
        plsc.subcore_barrier()
        pltpu.sync_copy(acc_sh.at[pl.ds(row0, RPT)], zbuf_v)
        pltpu.sync_copy(zbuf_v, out_hbm.at[pl.ds(c * N_ACC + row0, RPT)])

    return k(dst1d).reshape(NC, N_ACC, FD)


def _tc_prescale1(degp, x, W1):
    def body(degp_ref, x_ref, w1_ref, h1p_ref, dinv_ref):
        deg = degp_ref[0, :N, 0:1] + degp_ref[1, :N, 0:1] + 1.0
        dinv = lax.rsqrt(deg)
        dinv_ref[...] = dinv
        h1p_ref[...] = jnp.dot(x_ref[...], w1_ref[...],
                               preferred_element_type=jnp.float32) * dinv

    return pl.pallas_call(
        body,
        out_shape=[jax.ShapeDtypeStruct((N, 32), jnp.float32),
                   jax.ShapeDtypeStruct((N, 1), jnp.float32)],
    )(degp, x, W1)


def _tc_combine2(s1, h1p, dinv, W2, b1r):
    def body(s1_ref, h1p_ref, dinv_ref, w2_ref, b1_ref, h2p_ref):
        h1 = (_psum(s1_ref) + h1p_ref[...]) * dinv_ref[...] + b1_ref[...]
        h2p_ref[...] = jnp.dot(h1, w2_ref[...],
                               preferred_element_type=jnp.float32) * dinv_ref[...]

    return pl.pallas_call(
        body,
        out_shape=jax.ShapeDtypeStruct((N, 64), jnp.float32),
    )(s1, h1p, dinv, W2, b1r)


def _tc_head(s2, h2p, dinv, b2r, batch_row, fc1_W, fc1_br, bn_gr, bn_br,
             fc2_W, fc2_br):
    def body(s2_ref, h2p_ref, dinv_ref, b2_ref, bat_ref, w1_ref, bb1_ref,
             g_ref, bgb_ref, w2_ref, bb2_ref, out_ref):
        h2 = (_psum(s2_ref) + h2p_ref[...]) * dinv_ref[...] + b2_ref[...]
        gids = lax.broadcasted_iota(jnp.int32, (G, N), 0)
        oh = (gids == bat_ref[...]).astype(jnp.float32)
        sums = lax.dot_general(oh, h2, (((1,), (0,)), ((), ())),
                               preferred_element_type=jnp.float32)
        cnts = jnp.sum(oh, axis=1, keepdims=True)
        g = sums / jnp.maximum(cnts, 1.0)
        z = jnp.dot(g, w1_ref[...], preferred_element_type=jnp.float32) \
            + bb1_ref[...]
        z = jnp.maximum(z, 0.0)
        mu = jnp.mean(z, axis=0, keepdims=True)
        var = jnp.mean((z - mu) ** 2, axis=0, keepdims=True)
        z = (z - mu) * lax.rsqrt(var + 1e-5) * g_ref[...] + bgb_ref[...]
        o = jnp.dot(z, w2_ref[...], preferred_element_type=jnp.float32) \
            + bb2_ref[...]
        out_ref[...] = jnp.maximum(o, 0.0) + jnp.log1p(jnp.exp(-jnp.abs(o)))

    return pl.pallas_call(
        body,
        out_shape=jax.ShapeDtypeStruct((G, 1), jnp.float32),
    )(s2, h2p, dinv, b2r, batch_row, fc1_W, fc1_br, bn_gr, bn_br, fc2_W,
      fc2_br)


def kernel(x, W1, b1, W2, b2, fc1_W, fc1_b, bn_g, bn_b, fc2_W, fc2_b,
           edge_index, batch):
    src = edge_index[0]
    dst = edge_index[1]
    pad = E_PAD - E
    src1d = jnp.concatenate([src, jnp.zeros((pad,), jnp.int32)])
    dst1d = jnp.concatenate([dst, jnp.full((pad,), N, jnp.int32)])

    degp = _sc_degree(dst1d)
    h1p, dinv = _tc_prescale1(degp, x, W1)
    s1 = _sc_spmm(h1p, src1d, dst1d, 32, 256, 58, 4, 3, 2)
    h2p = _tc_combine2(s1, h1p, dinv, W2, b1.reshape(1, 32))
    s2 = _sc_spmm(h2p, src1d, dst1d, 64, 256, 62, 4, 3, 2)
    out = _tc_head(s2, h2p, dinv, b2.reshape(1, 64), batch.reshape(1, N),
                   fc1_W, fc1_b.reshape(1, 32), bn_g.reshape(1, 32),
                   bn_b.reshape(1, 32), fc2_W, fc2_b.reshape(1, 1))
    return out
